# baseline (speedup 1.0000x reference)
import functools

import jax
import jax.numpy as jnp
from jax import lax
from jax.experimental import pallas as pl
from jax.experimental.pallas import tpu as pltpu
from jax.experimental.pallas import tpu_sc as plsc

VOCAB = 1000000
EMBED_DIM = 32
COL_BLOCK = 81920
TV_GRID = 13
TV_PAD = TV_GRID * COL_BLOCK

NUM_CORES = 2
NUM_SUBCORES = 16
NUM_WORKERS = NUM_CORES * NUM_SUBCORES


def _tv_body(tabt_ref, w_ref, b_ref, out_ref):
    acc = jnp.sum(tabt_ref[...] * w_ref[...], axis=0)
    out_ref[...] = jax.nn.sigmoid(acc + b_ref[0, 0])


def _compute_tv(table, W, b):
    tabt = table.T
    b2 = b.reshape(1, 1)
    out = pl.pallas_call(
        _tv_body,
        grid=(TV_GRID,),
        in_specs=[
            pl.BlockSpec((EMBED_DIM, COL_BLOCK), lambda i: (0, i)),
            pl.BlockSpec((EMBED_DIM, 1), lambda i: (0, 0)),
            pl.BlockSpec(memory_space=pltpu.SMEM),
        ],
        out_specs=pl.BlockSpec((COL_BLOCK,), lambda i: (i,)),
        out_shape=jax.ShapeDtypeStruct((TV_PAD,), jnp.float32),
    )(tabt, W, b2)
    return out


def _make_gather(total):
    chunk = total // NUM_WORKERS
    tv_slice = TV_PAD // NUM_SUBCORES
    mesh = plsc.VectorSubcoreMesh(core_axis_name="c", subcore_axis_name="s")

    @functools.partial(
        pl.kernel,
        mesh=mesh,
        out_type=jax.ShapeDtypeStruct((total,), jnp.float32),
        scratch_types=[
            pltpu.VMEM((chunk,), jnp.int32),
            pltpu.VMEM((chunk,), jnp.float32),
            pltpu.VMEM_SHARED((TV_PAD,), jnp.float32),
            pltpu.SemaphoreType.DMA,
        ],
    )
    def gather(tv_hbm, idx_hbm, out_hbm, idx_v, val_v, tv_sh, sem):
        sid = lax.axis_index("s")
        wid = sid * NUM_CORES + lax.axis_index("c")
        base = wid * chunk
        pltpu.sync_copy(idx_hbm.at[pl.ds(base, chunk)], idx_v)
        off = sid * tv_slice
        pltpu.sync_copy(tv_hbm.at[pl.ds(off, tv_slice)],
                        tv_sh.at[pl.ds(off, tv_slice)])
        plsc.subcore_barrier()
        pltpu.async_copy(tv_sh.at[idx_v], val_v, sem).wait()
        pltpu.sync_copy(val_v, out_hbm.at[pl.ds(base, chunk)])

    return gather


def kernel(inputs, table, W, b):
    batch, hist = inputs.shape
    total = batch * hist
    idx = inputs.T.reshape(total).astype(jnp.int32)
    tv = _compute_tv(table, W, b)
    g = _make_gather(total)(tv, idx)
    return g.reshape(hist, batch, 1).transpose(1, 0, 2)
